# baseline (speedup 1.0000x reference)
import functools

import jax
import jax.numpy as jnp
from jax import lax
from jax.experimental import pallas as pl
from jax.experimental.pallas import tpu as pltpu
from jax.experimental.pallas import tpu_sc as plsc

B = 16384
D = 64
Y_LOW, Y_HIGH = 14.0, 30.0

_N_BIAS = 100000

def _vtake(v, perm):
    return lax.gather(
        v, perm[:, None],
        dimension_numbers=lax.GatherDimensionNumbers(
            offset_dims=(), collapsed_slice_dims=(0,), start_index_map=(0,)),
        slice_sizes=(1,),
        mode=lax.GatherScatterMode.PROMISE_IN_BOUNDS)


_NC = 2
_NS = 16
_NW = _NC * _NS
_CHUNK = B // _NW
_G = _CHUNK // 16


def _sc_kernel(sidx_hbm, pidx_hbm, sfac_hbm, pfac_hbm, bias_hbm,
               out_hbm, sidx_v, pidx_v, pbidx_v, sridx_v, pridx_v,
               srows_v, prows_v, sb_v, pb_v, out_v, sem):
    wid = lax.axis_index("s") * _NC + lax.axis_index("c")
    base = wid * _CHUNK

    pltpu.sync_copy(sidx_hbm.at[pl.ds(base, _CHUNK)], sidx_v)
    pltpu.sync_copy(pidx_hbm.at[pl.ds(base, _CHUNK)], pidx_v)

    def shift_body(g, _):
        sl = pl.ds(g * 16, 16)
        pbidx_v[sl] = pidx_v[sl] + _N_BIAS
        sridx_v[sl] = lax.shift_right_logical(sidx_v[sl], 1)
        pridx_v[sl] = lax.shift_right_logical(pidx_v[sl], 1)
        return 0

    lax.fori_loop(0, _G, shift_body, 0)

    c1 = pltpu.async_copy(sfac_hbm.at[sridx_v], srows_v, sem)
    c2 = pltpu.async_copy(pfac_hbm.at[pridx_v], prows_v, sem)
    c3 = pltpu.async_copy(bias_hbm.at[sidx_v], sb_v, sem)
    c4 = pltpu.async_copy(bias_hbm.at[pbidx_v], pb_v, sem)
    c1.wait()
    c2.wait()
    c3.wait()
    c4.wait()

    scale = jnp.full((16,), Y_HIGH - Y_LOW, jnp.float32)
    low = jnp.full((16,), Y_LOW, jnp.float32)
    lanes = lax.iota(jnp.int32, 16)
    perms = [lax.bitwise_xor(lanes, jnp.int32(1 << b)) for b in range(4)]
    lane_masks = [lanes == l for l in range(16)]

    def group_body(g, _):
        res = jnp.zeros((16,), jnp.float32)
        soffs = (sidx_v[pl.ds(g * 16, 16)] & 1) * 64
        poffs = (pidx_v[pl.ds(g * 16, 16)] & 1) * 64
        for l in range(16):
            j = g * 16 + l
            soff = soffs[l]
            poff = poffs[l]
            acc = jnp.zeros((16,), jnp.float32)
            for k in range(D // 32):
                s2 = srows_v[j, pl.ds(soff + k * 32, 32)]
                p2 = prows_v[j, pl.ds(poff + k * 32, 32)]
                se, so = plsc.unpack(s2, format=plsc.PackFormat.INTERLEAVED)
                pe, po = plsc.unpack(p2, format=plsc.PackFormat.INTERLEAVED)
                acc = acc + se * pe + so * po
            for perm in perms:
                acc = acc + _vtake(acc, perm)
            res = jnp.where(lane_masks[l], acc, res)
        sl = pl.ds(g * 16, 16)
        acc = res + sb_v[sl] + pb_v[sl]
        sig = 1.0 / (1.0 + jnp.exp(-acc))
        out_v[sl] = sig * scale + low
        return 0

    lax.fori_loop(0, _G, group_body, 0)

    pltpu.sync_copy(out_v, out_hbm.at[pl.ds(base, _CHUNK)])


@jax.jit
def _run(sidx, pidx, sample_factors, peptide_factors, bias_all):
    mesh = plsc.VectorSubcoreMesh(core_axis_name="c", subcore_axis_name="s")
    f = functools.partial(
        pl.kernel,
        out_type=jax.ShapeDtypeStruct((B,), jnp.float32),
        mesh=mesh,
        compiler_params=pltpu.CompilerParams(use_tc_tiling_on_sc=False,
                                             needs_layout_passes=False),
        scratch_types=[
            pltpu.VMEM((_CHUNK,), jnp.int32),
            pltpu.VMEM((_CHUNK,), jnp.int32),
            pltpu.VMEM((_CHUNK,), jnp.int32),
            pltpu.VMEM((_CHUNK,), jnp.int32),
            pltpu.VMEM((_CHUNK,), jnp.int32),
            pltpu.VMEM((_CHUNK, 2 * D), jnp.bfloat16),
            pltpu.VMEM((_CHUNK, 2 * D), jnp.bfloat16),
            pltpu.VMEM((_CHUNK,), jnp.float32),
            pltpu.VMEM((_CHUNK,), jnp.float32),
            pltpu.VMEM((_CHUNK,), jnp.float32),
            pltpu.SemaphoreType.DMA,
        ],
    )(_sc_kernel)
    return f(sidx, pidx, sample_factors, peptide_factors, bias_all)


@jax.jit
def kernel(x, sample_factors, sample_bias, peptide_factors, peptide_bias):
    bias_all = jnp.concatenate(
        [sample_bias.reshape(-1), peptide_bias.reshape(-1)])
    sf16 = sample_factors.astype(jnp.bfloat16).reshape(_N_BIAS // 2, 2 * D)
    pf16 = peptide_factors.astype(jnp.bfloat16).reshape(_N_BIAS // 2, 2 * D)
    res = _run(x[:, 0], x[:, 1], sf16, pf16, bias_all)
    return res.reshape(B, 1)


# speedup vs baseline: 1.3620x; 1.3620x over previous
import functools

import jax
import jax.numpy as jnp
from jax import lax
from jax.experimental import pallas as pl
from jax.experimental.pallas import tpu as pltpu
from jax.experimental.pallas import tpu_sc as plsc

B = 16384
D = 64
Y_LOW, Y_HIGH = 14.0, 30.0

_N_BIAS = 100000

def _vtake(v, perm):
    return lax.gather(
        v, perm[:, None],
        dimension_numbers=lax.GatherDimensionNumbers(
            offset_dims=(), collapsed_slice_dims=(0,), start_index_map=(0,)),
        slice_sizes=(1,),
        mode=lax.GatherScatterMode.PROMISE_IN_BOUNDS)


_NC = 2
_NS = 16
_NW = _NC * _NS
_CHUNK = B // _NW
_G = _CHUNK // 16


def _sc_kernel(sidx_hbm, pidx_hbm, sfac_hbm, pfac_hbm, bias_hbm,
               out_hbm, sidx_v, pidx_v, pbidx_v, srows_v, prows_v,
               sb_v, pb_v, out_v, sem):
    wid = lax.axis_index("s") * _NC + lax.axis_index("c")
    base = wid * _CHUNK

    pltpu.sync_copy(sidx_hbm.at[pl.ds(base, _CHUNK)], sidx_v)
    pltpu.sync_copy(pidx_hbm.at[pl.ds(base, _CHUNK)], pidx_v)

    def shift_body(g, _):
        pbidx_v[pl.ds(g * 16, 16)] = pidx_v[pl.ds(g * 16, 16)] + _N_BIAS
        return 0

    lax.fori_loop(0, _G, shift_body, 0)

    c1 = pltpu.async_copy(sfac_hbm.at[sidx_v], srows_v, sem)
    c2 = pltpu.async_copy(pfac_hbm.at[pidx_v], prows_v, sem)
    c3 = pltpu.async_copy(bias_hbm.at[sidx_v], sb_v, sem)
    c4 = pltpu.async_copy(bias_hbm.at[pbidx_v], pb_v, sem)
    c1.wait()
    c2.wait()
    c3.wait()
    c4.wait()

    scale = jnp.full((16,), Y_HIGH - Y_LOW, jnp.float32)
    low = jnp.full((16,), Y_LOW, jnp.float32)
    lanes = lax.iota(jnp.int32, 16)
    perms = [lax.bitwise_xor(lanes, jnp.int32(1 << b)) for b in range(4)]
    lane_masks = [lanes == l for l in range(16)]

    def group_body(g, _):
        res = jnp.zeros((16,), jnp.float32)
        for l in range(16):
            j = g * 16 + l
            acc = srows_v[j, pl.ds(0, 16)] * prows_v[j, pl.ds(0, 16)]
            for k in range(1, D // 16):
                acc = acc + (srows_v[j, pl.ds(k * 16, 16)] *
                             prows_v[j, pl.ds(k * 16, 16)])
            for perm in perms:
                acc = acc + _vtake(acc, perm)
            res = jnp.where(lane_masks[l], acc, res)
        sl = pl.ds(g * 16, 16)
        acc = res + sb_v[sl] + pb_v[sl]
        sig = 1.0 / (1.0 + jnp.exp(-acc))
        out_v[sl] = sig * scale + low
        return 0

    lax.fori_loop(0, _G, group_body, 0)

    pltpu.sync_copy(out_v, out_hbm.at[pl.ds(base, _CHUNK)])


@jax.jit
def _run(sidx, pidx, sample_factors, peptide_factors, bias_all):
    mesh = plsc.VectorSubcoreMesh(core_axis_name="c", subcore_axis_name="s")
    f = functools.partial(
        pl.kernel,
        out_type=jax.ShapeDtypeStruct((B,), jnp.float32),
        mesh=mesh,
        compiler_params=pltpu.CompilerParams(use_tc_tiling_on_sc=False,
                                             needs_layout_passes=False),
        scratch_types=[
            pltpu.VMEM((_CHUNK,), jnp.int32),
            pltpu.VMEM((_CHUNK,), jnp.int32),
            pltpu.VMEM((_CHUNK,), jnp.int32),
            pltpu.VMEM((_CHUNK, D), jnp.float32),
            pltpu.VMEM((_CHUNK, D), jnp.float32),
            pltpu.VMEM((_CHUNK,), jnp.float32),
            pltpu.VMEM((_CHUNK,), jnp.float32),
            pltpu.VMEM((_CHUNK,), jnp.float32),
            pltpu.SemaphoreType.DMA,
        ],
    )(_sc_kernel)
    return f(sidx, pidx, sample_factors, peptide_factors, bias_all)


@jax.jit
def kernel(x, sample_factors, sample_bias, peptide_factors, peptide_bias):
    bias_all = jnp.concatenate(
        [sample_bias.reshape(-1), peptide_bias.reshape(-1)])
    res = _run(x[:, 0], x[:, 1], sample_factors, peptide_factors, bias_all)
    return res.reshape(B, 1)
